# Initial kernel scaffold; baseline (speedup 1.0000x reference)
import jax
import jax.numpy as jnp
from jax.experimental import pallas as pl


def kernel(x, edge_index, W1, b1, W2, b2, Wl, bl):
    raise NotImplementedError("write your pallas kernel here")


# speedup vs baseline: 16.4590x; 16.4590x over previous
import functools

import jax
import jax.numpy as jnp
from jax import lax
from jax.experimental import pallas as pl
from jax.experimental.pallas import tpu as pltpu
from jax.experimental.pallas import tpu_sc as plsc

N_NODES = 100000
N_EDGES = 6400000
N_PAD = 100096
E_PAD = 6422528
PAD_E = E_PAD - N_EDGES
EROWS = E_PAD // 128
CHUNK = 25600
DUMP = 128
ACC2 = CHUNK + DUMP
R = 2000
GRID = N_NODES // R

_mesh = plsc.VectorSubcoreMesh(core_axis_name="c", subcore_axis_name="s")
_f32 = jnp.float32
_i32 = jnp.int32


@functools.partial(
    pl.kernel,
    out_type=jax.ShapeDtypeStruct((32, N_PAD), _f32),
    mesh=_mesh,
    compiler_params=pltpu.CompilerParams(needs_layout_passes=False),
    scratch_types=[
        pltpu.VMEM((N_PAD,), _f32),
        pltpu.VMEM((32, 128), _i32),
    ],
)
def _deg_sc(dst_hbm, out_hbm, acc, blk):
    c = lax.axis_index("c")
    s = lax.axis_index("s")
    wid = c * 16 + s
    zeros = jnp.zeros((16,), _f32)
    ones = jnp.ones((16,), _f32)

    def _zero(i, _):
        acc[pl.ds(i * 16, 16)] = zeros
        return _
    lax.fori_loop(0, N_PAD // 16, _zero, None)

    rows_per_tile = EROWS // 32
    base = wid * rows_per_tile

    def _block(b, _):
        pltpu.sync_copy(dst_hbm.at[pl.ds(base + b * 32, 32), :], blk)

        def _hist(j, _):
            g = j // 8
            k = j % 8
            idx = blk[g, pl.ds(k * 16, 16)]
            plsc.addupdate_scatter(acc, [idx], ones)
            return _
        lax.fori_loop(0, 32 * 8, _hist, None)
        return _
    lax.fori_loop(0, rows_per_tile // 32, _block, None)

    pltpu.sync_copy(acc, out_hbm.at[wid])


@functools.partial(
    pl.kernel,
    out_type=jax.ShapeDtypeStruct((2, N_PAD, 16), _f32),
    mesh=_mesh,
    compiler_params=pltpu.CompilerParams(
        needs_layout_passes=False, use_tc_tiling_on_sc=False),
    scratch_types=[
        pltpu.VMEM_SHARED((N_PAD, 16), _f32),
        pltpu.VMEM((8, 128), _i32),
        pltpu.VMEM((8, 128), _i32),
        pltpu.VMEM((1024, 16), _f32),
        pltpu.SemaphoreType.DMA,
    ],
)
def _seg1_sc(src_hbm, dst_hbm, u16_hbm, out_hbm, acc, sblk, dblk, rows, sem):
    c = lax.axis_index("c")
    s = lax.axis_index("s")
    zeros = jnp.zeros((16,), _f32)

    def _zb(i, _):
        rows[i, :] = zeros
        return _
    lax.fori_loop(0, 782, _zb, None)

    zr = s * (N_PAD // 16)

    def _za(k, _):
        pltpu.sync_copy(
            rows.at[pl.ds(0, 782), :], acc.at[pl.ds(zr + k * 782, 782), :])
        return _
    lax.fori_loop(0, 8, _za, None)
    plsc.subcore_barrier()

    rows_per_tile = EROWS // 32
    base = c * (EROWS // 2) + s * rows_per_tile

    def _block(b, _):
        r0 = base + b * 8
        pltpu.sync_copy(src_hbm.at[pl.ds(r0, 8), :], sblk)
        pltpu.sync_copy(dst_hbm.at[pl.ds(r0, 8), :], dblk)
        hs = [
            pltpu.async_copy(
                u16_hbm.at[sblk.at[g]], rows.at[pl.ds(g * 128, 128), :], sem)
            for g in range(8)
        ]
        for h in hs:
            h.wait()
        for g in range(8):
            pltpu.sync_copy(
                rows.at[pl.ds(g * 128, 128), :], acc.at[dblk.at[g]], add=True)
        return _
    lax.fori_loop(0, rows_per_tile // 8, _block, None)
    plsc.subcore_barrier()

    pltpu.sync_copy(
        acc.at[pl.ds(zr, N_PAD // 16), :],
        out_hbm.at[c, pl.ds(zr, N_PAD // 16), :])


@functools.partial(
    pl.kernel,
    out_type=jax.ShapeDtypeStruct((4 * CHUNK, 64), _f32),
    mesh=_mesh,
    compiler_params=pltpu.CompilerParams(
        needs_layout_passes=False, use_tc_tiling_on_sc=False),
    scratch_types=[
        pltpu.VMEM_SHARED((ACC2, 64), _f32),
        pltpu.VMEM((8, 128), _i32),
        pltpu.VMEM((8, 128), _i32),
        pltpu.VMEM((8, 128), _i32),
        pltpu.VMEM((256, 64), _f32),
        pltpu.SemaphoreType.DMA,
    ],
)
def _seg2_sc(src_hbm, dst_hbm, u2_hbm, out_hbm,
             acc, sblk, dblk, dlblk, rows, sem):
    c = lax.axis_index("c")
    s = lax.axis_index("s")
    zeros = jnp.zeros((16,), _f32)

    rows_per_tile = EROWS // 16
    base = s * rows_per_tile

    for p in range(2):
        chunk = c + 2 * p
        lo = chunk * CHUNK

        def _zb(i, _):
            g = i // 4
            k = i % 4
            rows[g, pl.ds(k * 16, 16)] = zeros
            return _
        lax.fori_loop(0, 256 * 4, _zb, None)

        def _za(k, _):
            pltpu.sync_copy(
                rows.at[pl.ds(0, 201), :],
                acc.at[pl.ds(s * (ACC2 // 16) + k * 201, 201), :])
            return _
        lax.fori_loop(0, 8, _za, None)
        plsc.subcore_barrier()

        def _block(b, _):
            r0 = base + b * 8
            pltpu.sync_copy(src_hbm.at[pl.ds(r0, 8), :], sblk)
            pltpu.sync_copy(dst_hbm.at[pl.ds(r0, 8), :], dblk)

            def _local(j, _):
                g = j // 8
                k = j % 8
                d = dblk[g, pl.ds(k * 16, 16)]
                inm = (d >= lo) & (d < lo + CHUNK)
                dl = jnp.where(inm, d - lo, CHUNK + (d & (DUMP - 1)))
                dlblk[g, pl.ds(k * 16, 16)] = dl
                return _
            lax.fori_loop(0, 64, _local, None)

            for sub in range(4):
                hs = [
                    pltpu.async_copy(
                        u2_hbm.at[sblk.at[2 * sub + q]],
                        rows.at[pl.ds(q * 128, 128), :], sem)
                    for q in range(2)
                ]
                for h in hs:
                    h.wait()
                for q in range(2):
                    pltpu.sync_copy(
                        rows.at[pl.ds(q * 128, 128), :],
                        acc.at[dlblk.at[2 * sub + q]], add=True)
            return _
        lax.fori_loop(0, rows_per_tile // 8, _block, None)
        plsc.subcore_barrier()

        pltpu.sync_copy(
            acc.at[pl.ds(s * (CHUNK // 16), CHUNK // 16), :],
            out_hbm.at[pl.ds(lo + s * (CHUNK // 16), CHUNK // 16), :])
        plsc.subcore_barrier()


def _stage_a0(dp_ref, deg_ref):
    deg_ref[...] = jnp.sum(dp_ref[...], axis=0)[:, None]


def _stage_a(deg_ref, x_ref, dis_ref, u16_ref):
    deg = deg_ref[...][:, 0] + 1.0
    dis = lax.rsqrt(deg)
    dis_ref[...] = dis[:, None]
    u16_ref[...] = jnp.concatenate(
        [dis[:, None] * x_ref[...], jnp.zeros((R, 14), _f32)], axis=1)


def _stage_b(t1_ref, u16_ref, dis_ref, w1_ref, b1_ref, w2_ref, u2_ref):
    t1 = t1_ref[0] + t1_ref[1]
    dis = dis_ref[...]
    s1 = dis * (t1[:, :2] + u16_ref[..., :2])
    h1 = jnp.maximum(
        jnp.dot(s1, w1_ref[...], preferred_element_type=_f32) + b1_ref[...],
        0.0)
    z = jnp.dot(h1, w2_ref[...], preferred_element_type=_f32)
    u2_ref[...] = dis * z


def _stage_c(t2_ref, u2_ref, dis_ref, b2_ref, wl_ref, bl_ref, out_ref, acc):
    i = pl.program_id(0)
    h2 = jnp.maximum(
        dis_ref[...] * (t2_ref[...] + u2_ref[...]) + b2_ref[...], 0.0)
    part = jnp.sum(h2, axis=0, keepdims=True)

    @pl.when(i == 0)
    def _():
        acc[...] = part

    @pl.when(i > 0)
    def _():
        acc[...] = acc[...] + part

    @pl.when(i == GRID - 1)
    def _():
        pooled = acc[...] / float(N_NODES)
        out_ref[...] = (
            jnp.dot(pooled, wl_ref[...], preferred_element_type=_f32)
            + bl_ref[...])


def kernel(x, edge_index, W1, b1, W2, b2, Wl, bl):
    ei = edge_index.astype(_i32)
    src = jnp.concatenate([ei[0], jnp.zeros((PAD_E,), _i32)])
    dst = jnp.concatenate([ei[1], jnp.full((PAD_E,), N_NODES, _i32)])
    src2d = src.reshape(EROWS, 128)
    dst2d = dst.reshape(EROWS, 128)

    degpart = _deg_sc(dst2d)

    deg2d = pl.pallas_call(
        _stage_a0,
        grid=(17,),
        in_specs=[pl.BlockSpec((32, 5888), lambda i: (0, i))],
        out_specs=pl.BlockSpec((5888, 1), lambda i: (i, 0)),
        out_shape=jax.ShapeDtypeStruct((N_PAD, 1), _f32),
    )(degpart)

    dis, u16 = pl.pallas_call(
        _stage_a,
        grid=(GRID,),
        in_specs=[
            pl.BlockSpec((R, 1), lambda i: (i, 0)),
            pl.BlockSpec((R, 2), lambda i: (i, 0)),
        ],
        out_specs=[
            pl.BlockSpec((R, 1), lambda i: (i, 0)),
            pl.BlockSpec((R, 16), lambda i: (i, 0)),
        ],
        out_shape=[
            jax.ShapeDtypeStruct((N_NODES, 1), _f32),
            jax.ShapeDtypeStruct((N_NODES, 16), _f32),
        ],
    )(deg2d, x)

    t1part = _seg1_sc(src2d, dst2d, u16)

    u2 = pl.pallas_call(
        _stage_b,
        grid=(GRID,),
        in_specs=[
            pl.BlockSpec((2, R, 16), lambda i: (0, i, 0)),
            pl.BlockSpec((R, 16), lambda i: (i, 0)),
            pl.BlockSpec((R, 1), lambda i: (i, 0)),
            pl.BlockSpec((2, 64), lambda i: (0, 0)),
            pl.BlockSpec((1, 64), lambda i: (0, 0)),
            pl.BlockSpec((64, 64), lambda i: (0, 0)),
        ],
        out_specs=pl.BlockSpec((R, 64), lambda i: (i, 0)),
        out_shape=jax.ShapeDtypeStruct((N_NODES, 64), _f32),
    )(t1part, u16, dis, W1, b1.reshape(1, 64), W2)

    t2 = _seg2_sc(src2d, dst2d, u2)

    out = pl.pallas_call(
        _stage_c,
        grid=(GRID,),
        in_specs=[
            pl.BlockSpec((R, 64), lambda i: (i, 0)),
            pl.BlockSpec((R, 64), lambda i: (i, 0)),
            pl.BlockSpec((R, 1), lambda i: (i, 0)),
            pl.BlockSpec((1, 64), lambda i: (0, 0)),
            pl.BlockSpec((64, 1), lambda i: (0, 0)),
            pl.BlockSpec((1, 1), lambda i: (0, 0)),
        ],
        out_specs=pl.BlockSpec((1, 1), lambda i: (0, 0)),
        out_shape=jax.ShapeDtypeStruct((1, 1), _f32),
        scratch_shapes=[pltpu.VMEM((1, 64), _f32)],
    )(t2, u2, dis, b2.reshape(1, 64), Wl, bl.reshape(1, 1))

    return out.reshape(1)


# speedup vs baseline: 33.7101x; 2.0481x over previous
import functools

import jax
import jax.numpy as jnp
from jax import lax
from jax.experimental import pallas as pl
from jax.experimental.pallas import tpu as pltpu
from jax.experimental.pallas import tpu_sc as plsc

N_NODES = 100000
N_EDGES = 6400000
N_PAD = 100096
E_PAD = 6422528
PAD_E = E_PAD - N_EDGES
EROWS = E_PAD // 128
CHUNK = 25600
ACC2 = CHUNK + 8
CAP = 1280
R = 2000
GRID = N_NODES // R

_mesh = plsc.VectorSubcoreMesh(core_axis_name="c", subcore_axis_name="s")
_f32 = jnp.float32
_i32 = jnp.int32


@functools.partial(
    pl.kernel,
    out_type=jax.ShapeDtypeStruct((32, N_PAD), _f32),
    mesh=_mesh,
    compiler_params=pltpu.CompilerParams(needs_layout_passes=False),
    scratch_types=[
        pltpu.VMEM((N_PAD,), _f32),
        pltpu.VMEM((32, 128), _i32),
    ],
)
def _deg_sc(dst_hbm, out_hbm, acc, blk):
    c = lax.axis_index("c")
    s = lax.axis_index("s")
    wid = c * 16 + s
    zeros = jnp.zeros((16,), _f32)
    ones = jnp.ones((16,), _f32)

    def _zero(i, _):
        acc[pl.ds(i * 16, 16)] = zeros
        return _
    lax.fori_loop(0, N_PAD // 16, _zero, None)

    rows_per_tile = EROWS // 32
    base = wid * rows_per_tile

    def _block(b, _):
        pltpu.sync_copy(dst_hbm.at[pl.ds(base + b * 32, 32), :], blk)

        def _hist(j, _):
            g = j // 8
            k = j % 8
            idx = blk[g, pl.ds(k * 16, 16)]
            plsc.addupdate_scatter(acc, [idx], ones)
            return _
        lax.fori_loop(0, 32 * 8, _hist, None)
        return _
    lax.fori_loop(0, rows_per_tile // 32, _block, None)

    pltpu.sync_copy(acc, out_hbm.at[wid])


@functools.partial(
    pl.kernel,
    out_type=jax.ShapeDtypeStruct((2, N_PAD, 16), _f32),
    mesh=_mesh,
    compiler_params=pltpu.CompilerParams(
        needs_layout_passes=False, use_tc_tiling_on_sc=False),
    scratch_types=[
        pltpu.VMEM_SHARED((N_PAD, 16), _f32),
        pltpu.VMEM((8, 128), _i32),
        pltpu.VMEM((8, 128), _i32),
        pltpu.VMEM((1024, 16), _f32),
        pltpu.SemaphoreType.DMA,
    ],
)
def _seg1_sc(src_hbm, dst_hbm, u16_hbm, out_hbm, acc, sblk, dblk, rows, sem):
    c = lax.axis_index("c")
    s = lax.axis_index("s")
    zeros = jnp.zeros((16,), _f32)

    def _zb(i, _):
        rows[i, :] = zeros
        return _
    lax.fori_loop(0, 782, _zb, None)

    zr = s * (N_PAD // 16)

    def _za(k, _):
        pltpu.sync_copy(
            rows.at[pl.ds(0, 782), :], acc.at[pl.ds(zr + k * 782, 782), :])
        return _
    lax.fori_loop(0, 8, _za, None)
    plsc.subcore_barrier()

    rows_per_tile = EROWS // 32
    base = c * (EROWS // 2) + s * rows_per_tile

    def _block(b, _):
        r0 = base + b * 8
        pltpu.sync_copy(src_hbm.at[pl.ds(r0, 8), :], sblk)
        pltpu.sync_copy(dst_hbm.at[pl.ds(r0, 8), :], dblk)
        hs = [
            pltpu.async_copy(
                u16_hbm.at[sblk.at[g]], rows.at[pl.ds(g * 128, 128), :], sem)
            for g in range(8)
        ]
        for h in hs:
            h.wait()
        for g in range(8):
            pltpu.sync_copy(
                rows.at[pl.ds(g * 128, 128), :], acc.at[dblk.at[g]], add=True)
        return _
    lax.fori_loop(0, rows_per_tile // 8, _block, None)
    plsc.subcore_barrier()

    pltpu.sync_copy(
        acc.at[pl.ds(zr, N_PAD // 16), :],
        out_hbm.at[c, pl.ds(zr, N_PAD // 16), :])


@functools.partial(
    pl.kernel,
    out_type=jax.ShapeDtypeStruct((4 * CHUNK, 64), _f32),
    mesh=_mesh,
    compiler_params=pltpu.CompilerParams(
        needs_layout_passes=False, use_tc_tiling_on_sc=False),
    scratch_types=[
        pltpu.VMEM_SHARED((ACC2, 64), _f32),
        pltpu.VMEM((8, 128), _i32),
        pltpu.VMEM((8, 128), _i32),
        pltpu.VMEM((CAP,), _i32),
        pltpu.VMEM((CAP,), _i32),
        pltpu.VMEM((2, 128), _i32),
        pltpu.VMEM((256, 64), _f32),
        pltpu.SemaphoreType.DMA,
    ],
)
def _seg2_sc(src_hbm, dst_hbm, u2_hbm, out_hbm,
             acc, sblk, dblk, csrc, cdst, didx, rowbuf, sem):
    c = lax.axis_index("c")
    s = lax.axis_index("s")
    zeros = jnp.zeros((16,), _f32)

    rows_per_tile = EROWS // 16
    base = s * rows_per_tile

    for p in range(2):
        lo = (c + 2 * p) * CHUNK

        def _zb(i, _):
            g = i // 4
            k = i % 4
            rowbuf[g, pl.ds(k * 16, 16)] = zeros
            return _
        lax.fori_loop(0, 64 * 4, _zb, None)

        def _za(k, _):
            pltpu.sync_copy(
                rowbuf.at[pl.ds(0, 64), :],
                acc.at[pl.ds(s * (CHUNK // 16) + k * 64, 64), :])
            return _
        lax.fori_loop(0, 25, _za, None)
        plsc.subcore_barrier()

        def _block(b, cnt):
            r0 = base + b * 8
            pltpu.sync_copy(src_hbm.at[pl.ds(r0, 8), :], sblk)
            pltpu.sync_copy(dst_hbm.at[pl.ds(r0, 8), :], dblk)

            def _comp(j, cnt):
                g = j // 8
                k = j % 8
                d = dblk[g, pl.ds(k * 16, 16)]
                sv = sblk[g, pl.ds(k * 16, 16)]
                inm = (d >= lo) & (d < lo + CHUNK)
                plsc.store_compressed(csrc.at[pl.ds(cnt, 16)], sv, mask=inm)
                plsc.store_compressed(cdst.at[pl.ds(cnt, 16)], d - lo, mask=inm)
                pcv = plsc.all_reduce_population_count(inm)
                pc = pcv if pcv.ndim == 0 else pcv[0]
                return cnt + pc
            cnt = lax.fori_loop(0, 64, _comp, cnt)

            nfull = lax.shift_right_logical(cnt, 7)

            def _pair(rp, _):
                offa = pl.multiple_of(rp * 256, 128)
                offb = offa + 128
                ha = pltpu.async_copy(
                    u2_hbm.at[csrc.at[pl.ds(offa, 128)]],
                    rowbuf.at[pl.ds(0, 128), :], sem)
                hb = pltpu.async_copy(
                    u2_hbm.at[csrc.at[pl.ds(offb, 128)]],
                    rowbuf.at[pl.ds(128, 128), :], sem)

                def _fd(j, _):
                    didx[0, pl.ds(j * 16, 16)] = cdst[pl.ds(offa + j * 16, 16)]
                    didx[1, pl.ds(j * 16, 16)] = cdst[pl.ds(offb + j * 16, 16)]
                    return _
                lax.fori_loop(0, 8, _fd, None)
                ha.wait()
                pltpu.sync_copy(
                    rowbuf.at[pl.ds(0, 128), :], acc.at[didx.at[0]], add=True)
                hb.wait()
                pltpu.sync_copy(
                    rowbuf.at[pl.ds(128, 128), :], acc.at[didx.at[1]], add=True)
                return _
            lax.fori_loop(0, lax.shift_right_logical(nfull, 1), _pair, None)

            @pl.when((nfull & 1) == 1)
            def _tail():
                offt = pl.multiple_of((nfull - 1) * 128, 128)
                ht = pltpu.async_copy(
                    u2_hbm.at[csrc.at[pl.ds(offt, 128)]],
                    rowbuf.at[pl.ds(0, 128), :], sem)

                def _fd(j, _):
                    didx[0, pl.ds(j * 16, 16)] = cdst[pl.ds(offt + j * 16, 16)]
                    return _
                lax.fori_loop(0, 8, _fd, None)
                ht.wait()
                pltpu.sync_copy(
                    rowbuf.at[pl.ds(0, 128), :], acc.at[didx.at[0]], add=True)

            @pl.when(nfull > 0)
            def _mv():
                def _mvv(j, _):
                    csrc[pl.ds(j * 16, 16)] = csrc[pl.ds(nfull * 128 + j * 16, 16)]
                    cdst[pl.ds(j * 16, 16)] = cdst[pl.ds(nfull * 128 + j * 16, 16)]
                    return _
                lax.fori_loop(0, 8, _mvv, None)
            return cnt & 127
        cnt = lax.fori_loop(0, rows_per_tile // 8, _block, jnp.int32(0))

        @pl.when(cnt > 0)
        def _flush():
            def _pad(j, _):
                csrc[pl.ds(cnt + j * 16, 16)] = jnp.zeros((16,), _i32)
                cdst[pl.ds(cnt + j * 16, 16)] = jnp.full((16,), CHUNK, _i32)
                return _
            lax.fori_loop(0, 8, _pad, None)
            hf = pltpu.async_copy(
                u2_hbm.at[csrc.at[pl.ds(0, 128)]],
                rowbuf.at[pl.ds(0, 128), :], sem)

            def _fd(j, _):
                didx[0, pl.ds(j * 16, 16)] = cdst[pl.ds(j * 16, 16)]
                return _
            lax.fori_loop(0, 8, _fd, None)
            hf.wait()
            pltpu.sync_copy(
                rowbuf.at[pl.ds(0, 128), :], acc.at[didx.at[0]], add=True)

        plsc.subcore_barrier()
        pltpu.sync_copy(
            acc.at[pl.ds(s * (CHUNK // 16), CHUNK // 16), :],
            out_hbm.at[pl.ds(lo + s * (CHUNK // 16), CHUNK // 16), :])
        plsc.subcore_barrier()


def _stage_a0(dp_ref, deg_ref):
    deg_ref[...] = jnp.sum(dp_ref[...], axis=0)[:, None]


def _stage_a(deg_ref, x_ref, dis_ref, u16_ref):
    deg = deg_ref[...][:, 0] + 1.0
    dis = lax.rsqrt(deg)
    dis_ref[...] = dis[:, None]
    u16_ref[...] = jnp.concatenate(
        [dis[:, None] * x_ref[...], jnp.zeros((R, 14), _f32)], axis=1)


def _stage_b(t1_ref, u16_ref, dis_ref, w1_ref, b1_ref, w2_ref, u2_ref):
    t1 = t1_ref[0] + t1_ref[1]
    dis = dis_ref[...]
    s1 = dis * (t1[:, :2] + u16_ref[..., :2])
    h1 = jnp.maximum(
        jnp.dot(s1, w1_ref[...], preferred_element_type=_f32) + b1_ref[...],
        0.0)
    z = jnp.dot(h1, w2_ref[...], preferred_element_type=_f32)
    u2_ref[...] = dis * z


def _stage_c(t2_ref, u2_ref, dis_ref, b2_ref, wl_ref, bl_ref, out_ref, acc):
    i = pl.program_id(0)
    h2 = jnp.maximum(
        dis_ref[...] * (t2_ref[...] + u2_ref[...]) + b2_ref[...], 0.0)
    part = jnp.sum(h2, axis=0, keepdims=True)

    @pl.when(i == 0)
    def _():
        acc[...] = part

    @pl.when(i > 0)
    def _():
        acc[...] = acc[...] + part

    @pl.when(i == GRID - 1)
    def _():
        pooled = acc[...] / float(N_NODES)
        out_ref[...] = (
            jnp.dot(pooled, wl_ref[...], preferred_element_type=_f32)
            + bl_ref[...])


def kernel(x, edge_index, W1, b1, W2, b2, Wl, bl):
    ei = edge_index.astype(_i32)
    src = jnp.concatenate([ei[0], jnp.zeros((PAD_E,), _i32)])
    dst = jnp.concatenate([ei[1], jnp.full((PAD_E,), N_NODES, _i32)])
    src2d = src.reshape(EROWS, 128)
    dst2d = dst.reshape(EROWS, 128)

    degpart = _deg_sc(dst2d)

    deg2d = pl.pallas_call(
        _stage_a0,
        grid=(17,),
        in_specs=[pl.BlockSpec((32, 5888), lambda i: (0, i))],
        out_specs=pl.BlockSpec((5888, 1), lambda i: (i, 0)),
        out_shape=jax.ShapeDtypeStruct((N_PAD, 1), _f32),
    )(degpart)

    dis, u16 = pl.pallas_call(
        _stage_a,
        grid=(GRID,),
        in_specs=[
            pl.BlockSpec((R, 1), lambda i: (i, 0)),
            pl.BlockSpec((R, 2), lambda i: (i, 0)),
        ],
        out_specs=[
            pl.BlockSpec((R, 1), lambda i: (i, 0)),
            pl.BlockSpec((R, 16), lambda i: (i, 0)),
        ],
        out_shape=[
            jax.ShapeDtypeStruct((N_NODES, 1), _f32),
            jax.ShapeDtypeStruct((N_NODES, 16), _f32),
        ],
    )(deg2d, x)

    t1part = _seg1_sc(src2d, dst2d, u16)

    u2 = pl.pallas_call(
        _stage_b,
        grid=(GRID,),
        in_specs=[
            pl.BlockSpec((2, R, 16), lambda i: (0, i, 0)),
            pl.BlockSpec((R, 16), lambda i: (i, 0)),
            pl.BlockSpec((R, 1), lambda i: (i, 0)),
            pl.BlockSpec((2, 64), lambda i: (0, 0)),
            pl.BlockSpec((1, 64), lambda i: (0, 0)),
            pl.BlockSpec((64, 64), lambda i: (0, 0)),
        ],
        out_specs=pl.BlockSpec((R, 64), lambda i: (i, 0)),
        out_shape=jax.ShapeDtypeStruct((N_NODES, 64), _f32),
    )(t1part, u16, dis, W1, b1.reshape(1, 64), W2)

    t2 = _seg2_sc(src2d, dst2d, u2)

    out = pl.pallas_call(
        _stage_c,
        grid=(GRID,),
        in_specs=[
            pl.BlockSpec((R, 64), lambda i: (i, 0)),
            pl.BlockSpec((R, 64), lambda i: (i, 0)),
            pl.BlockSpec((R, 1), lambda i: (i, 0)),
            pl.BlockSpec((1, 64), lambda i: (0, 0)),
            pl.BlockSpec((64, 1), lambda i: (0, 0)),
            pl.BlockSpec((1, 1), lambda i: (0, 0)),
        ],
        out_specs=pl.BlockSpec((1, 1), lambda i: (0, 0)),
        out_shape=jax.ShapeDtypeStruct((1, 1), _f32),
        scratch_shapes=[pltpu.VMEM((1, 64), _f32)],
    )(t2, u2, dis, b2.reshape(1, 64), Wl, bl.reshape(1, 1))

    return out.reshape(1)
